# baseline (speedup 1.0000x reference)
import functools

import jax
import jax.numpy as jnp
from jax import lax
from jax.experimental import pallas as pl
from jax.experimental.pallas import tpu as pltpu
from jax.experimental.pallas import tpu_sc as plsc

N_ATOM = 10000
N_FRAG = 10000
E = 160000
F = 128

NC = 2
NS = 16
CHUNK = 128
IDXG = 8
CHUNKS_PER_SUB = 80
EP = NS * CHUNK * CHUNKS_PER_SUB
NFP = 10240
ROWS_PER_SUB = NFP // NS
BLOCKS_PER_SUB = ROWS_PER_SUB // CHUNK


def _sc_agg(h, src, dst):
  mesh = plsc.VectorSubcoreMesh(core_axis_name="c", subcore_axis_name="s")

  @functools.partial(
      pl.kernel,
      mesh=mesh,
      out_type=jax.ShapeDtypeStruct((NC, NFP, F), jnp.float32),
      scratch_types=[
          pltpu.VMEM((2, IDXG, CHUNK), jnp.int32),
          pltpu.VMEM((2, IDXG, CHUNK), jnp.int32),
          pltpu.VMEM((CHUNK, F), jnp.float32),
          pltpu.VMEM((CHUNK, F), jnp.float32),
          pltpu.VMEM((CHUNK,), jnp.float32),
          pltpu.VMEM((ROWS_PER_SUB,), jnp.float32),
          pltpu.VMEM_SHARED((NFP, F), jnp.float32),
          pltpu.VMEM_SHARED((NFP,), jnp.float32),
          pltpu.SemaphoreType.DMA,
          pltpu.SemaphoreType.DMA,
          pltpu.SemaphoreType.DMA,
          pltpu.SemaphoreType.DMA,
      ],
  )
  def k(h_hbm, src_hbm, dst_hbm, means_hbm,
        src_v, dst_v, rows_v, rows_w, ones_v, cntb_v, accum, cnt_sp,
        sem, sem_s, sem_c, sem_i):
    c = lax.axis_index("c")
    s = lax.axis_index("s")
    base = s * ROWS_PER_SUB

    def fill_body(i, carry):
      for t in range(F // 16):
        rows_v[i, pl.ds(t * 16, 16)] = jnp.zeros((16,), jnp.float32)
      return carry
    lax.fori_loop(0, CHUNK, fill_body, 0)
    for t in range(CHUNK // 16):
      ones_v[pl.ds(t * 16, 16)] = jnp.ones((16,), jnp.float32)
    def zfill_body(i, carry):
      cntb_v[pl.ds(i * 16, 16)] = jnp.zeros((16,), jnp.float32)
      return carry
    lax.fori_loop(0, ROWS_PER_SUB // 16, zfill_body, 0)

    for t in range(BLOCKS_PER_SUB):
      pltpu.sync_copy(rows_v, accum.at[pl.ds(base + t * CHUNK, CHUNK)])
    pltpu.sync_copy(cntb_v, cnt_sp.at[pl.ds(base, ROWS_PER_SUB)])

    plsc.subcore_barrier()

    bufs = (rows_v, rows_w)
    ngroups = CHUNKS_PER_SUB // IDXG
    HALF = CHUNK // 2

    def gather2(idx_row, buf, gsem):
      pltpu.async_copy(h_hbm.at[idx_row.at[pl.ds(0, HALF)]],
                       buf.at[pl.ds(0, HALF)], gsem)
      pltpu.async_copy(h_hbm.at[idx_row.at[pl.ds(HALF, HALF)]],
                       buf.at[pl.ds(HALF, HALF)], gsem)

    def idx_load(g, p):
      pltpu.async_copy(
          src_hbm.at[c, pl.ds(s * CHUNKS_PER_SUB + g * IDXG, IDXG)],
          src_v.at[p], sem_i)
      pltpu.async_copy(
          dst_hbm.at[c, pl.ds(s * CHUNKS_PER_SUB + g * IDXG, IDXG)],
          dst_v.at[p], sem_i)

    def idx_drain(p):
      pltpu.make_async_copy(
          src_hbm.at[c, pl.ds(s * CHUNKS_PER_SUB, IDXG)],
          src_v.at[p], sem_i).wait()
      pltpu.make_async_copy(
          dst_hbm.at[c, pl.ds(s * CHUNKS_PER_SUB, IDXG)],
          dst_v.at[p], sem_i).wait()

    idx_load(0, 0)
    idx_drain(0)
    gather2(src_v.at[0, 0], bufs[0], sem)

    def chunk_body(g, carry):
      p = lax.rem(g, 2)
      @pl.when(g < ngroups - 1)
      def _():
        idx_load(g + 1, 1 - p)
      hc = [pltpu.async_copy(ones_v, cnt_sp.at[dst_v.at[p, t]], sem_c,
                             add=True)
            for t in range(IDXG)]
      hs = [None] * IDXG
      for t in range(IDXG):
        pltpu.make_async_copy(
            h_hbm.at[src_v.at[p, t]], bufs[t % 2], sem).wait()
        if t >= 1:
          hs[t - 1].wait()
        if t + 1 < IDXG:
          gather2(src_v.at[p, t + 1], bufs[(t + 1) % 2], sem)
        hs[t] = pltpu.async_copy(bufs[t % 2], accum.at[dst_v.at[p, t]],
                                 sem_s, add=True)
      hs[IDXG - 1].wait()
      for t in range(IDXG):
        hc[t].wait()
      @pl.when(g < ngroups - 1)
      def _():
        idx_drain(1 - p)
        gather2(src_v.at[1 - p, 0], bufs[0], sem)
      return carry
    lax.fori_loop(0, ngroups, chunk_body, 0)

    plsc.subcore_barrier()

    pltpu.sync_copy(cnt_sp.at[pl.ds(base, ROWS_PER_SUB)], cntb_v)

    def divide(buf, t):
      def div_group(g, carry2):
        cnt16 = cntb_v[pl.ds(t * CHUNK + g * 16, 16)]
        winv = 1.0 / jnp.maximum(cnt16, 1.0)
        for lane in range(16):
          b = jnp.full((16,), winv[lane], jnp.float32)
          r = g * 16 + lane
          for u in range(F // 16):
            buf[r, pl.ds(u * 16, 16)] = buf[r, pl.ds(u * 16, 16)] * b
        return carry2
      lax.fori_loop(0, CHUNK // 16, div_group, 0)

    hin = pltpu.async_copy(accum.at[pl.ds(base, CHUNK)], bufs[0], sem)
    hout = [None] * BLOCKS_PER_SUB
    for t in range(BLOCKS_PER_SUB):
      hin.wait()
      if t >= 1:
        hout[t - 1].wait()
      if t + 1 < BLOCKS_PER_SUB:
        hin = pltpu.async_copy(
            accum.at[pl.ds(base + (t + 1) * CHUNK, CHUNK)],
            bufs[(t + 1) % 2], sem)
      divide(bufs[t % 2], t)
      hout[t] = pltpu.async_copy(
          bufs[t % 2], means_hbm.at[c, pl.ds(base + t * CHUNK, CHUNK)], sem_s)
    hout[BLOCKS_PER_SUB - 1].wait()

  return k(h, src, dst)


def _tc_body(m_ref, w0, b0, w1, b1, w2, b2, w3, b3, w4, b4, o_ref):
  hf = m_ref[0] + m_ref[1]
  x = jnp.tanh(jnp.dot(hf, w0[...], preferred_element_type=jnp.float32)
               + b0[...])
  x = jnp.maximum(jnp.dot(x, w1[...], preferred_element_type=jnp.float32)
                  + b1[...], 0.0)
  x = jnp.maximum(jnp.dot(x, w2[...], preferred_element_type=jnp.float32)
                  + b2[...], 0.0)
  x = jnp.maximum(jnp.dot(x, w3[...], preferred_element_type=jnp.float32)
                  + b3[...], 0.0)
  o_ref[...] = jnp.dot(x, w4[...], preferred_element_type=jnp.float32) + b4[...]


def _tc_mlp(means, W0, b0, W1, b1, W2, b2, W3, b3, W4, b4):
  rows = 1024
  grid = (NFP // rows,)
  wspec = lambda shape: pl.BlockSpec(shape, lambda i: (0, 0))
  return pl.pallas_call(
      _tc_body,
      grid=grid,
      in_specs=[
          pl.BlockSpec((NC, rows, F), lambda i: (0, i, 0)),
          wspec((F, F)), wspec((1, F)),
          wspec((F, F)), wspec((1, F)),
          wspec((F, F)), wspec((1, F)),
          wspec((F, F)), wspec((1, F)),
          wspec((F, 1)), wspec((1, 1)),
      ],
      out_specs=pl.BlockSpec((rows, 1), lambda i: (i, 0)),
      out_shape=jax.ShapeDtypeStruct((NFP, 1), jnp.float32),
  )(means, W0, b0.reshape(1, F), W1, b1.reshape(1, F),
    W2, b2.reshape(1, F), W3, b3.reshape(1, F), W4, b4.reshape(1, 1))


def kernel(h, edge_index_0, edge_index_1,
           W0, b0, W1, b1, W2, b2, W3, b3, W4, b4):
  pad = EP - E
  pad_src = (jnp.arange(pad, dtype=jnp.int32) * 131) % N_ATOM
  pad_dst = N_FRAG + (jnp.arange(pad, dtype=jnp.int32) % (NFP - N_FRAG))
  src = jnp.stack([
      jnp.concatenate([edge_index_0[0], pad_src]),
      jnp.concatenate([edge_index_1[0], pad_src]),
  ]).reshape(2, EP // CHUNK, CHUNK)
  dst = jnp.stack([
      jnp.concatenate([edge_index_0[1], pad_dst]),
      jnp.concatenate([edge_index_1[1], pad_dst]),
  ]).reshape(2, EP // CHUNK, CHUNK)
  means = _sc_agg(h, src, dst)
  out = _tc_mlp(means, W0, b0, W1, b1, W2, b2, W3, b3, W4, b4)
  return out[:N_FRAG]


# speedup vs baseline: 1.0076x; 1.0076x over previous
import functools

import jax
import jax.numpy as jnp
from jax import lax
from jax.experimental import pallas as pl
from jax.experimental.pallas import tpu as pltpu
from jax.experimental.pallas import tpu_sc as plsc

N_ATOM = 10000
N_FRAG = 10000
E = 160000
F = 128

NC = 2
NS = 16
CHUNK = 128
IDXG = 8
CHUNKS_PER_SUB = 80
EP = NS * CHUNK * CHUNKS_PER_SUB
NFP = 10240
ROWS_PER_SUB = NFP // NS
BLOCKS_PER_SUB = ROWS_PER_SUB // CHUNK


def _sc_agg(h, src, dst):
  mesh = plsc.VectorSubcoreMesh(core_axis_name="c", subcore_axis_name="s")

  @functools.partial(
      pl.kernel,
      mesh=mesh,
      out_type=jax.ShapeDtypeStruct((NC, NFP, F), jnp.float32),
      scratch_types=[
          pltpu.VMEM((2, IDXG, CHUNK), jnp.int32),
          pltpu.VMEM((2, IDXG, CHUNK), jnp.int32),
          pltpu.VMEM((CHUNK, F), jnp.float32),
          pltpu.VMEM((CHUNK, F), jnp.float32),
          pltpu.VMEM((CHUNK,), jnp.float32),
          pltpu.VMEM((ROWS_PER_SUB,), jnp.float32),
          pltpu.VMEM_SHARED((NFP, F), jnp.float32),
          pltpu.VMEM_SHARED((NFP,), jnp.float32),
          pltpu.SemaphoreType.DMA,
          pltpu.SemaphoreType.DMA,
          pltpu.SemaphoreType.DMA,
          pltpu.SemaphoreType.DMA,
      ],
  )
  def k(h_hbm, src_hbm, dst_hbm, means_hbm,
        src_v, dst_v, rows_v, rows_w, ones_v, cntb_v, accum, cnt_sp,
        sem, sem_s, sem_c, sem_i):
    c = lax.axis_index("c")
    s = lax.axis_index("s")
    base = s * ROWS_PER_SUB
    bufs = (rows_w, rows_v)

    pltpu.async_copy(
        src_hbm.at[c, pl.ds(s * CHUNKS_PER_SUB, IDXG)], src_v.at[0], sem_i)
    pltpu.async_copy(
        dst_hbm.at[c, pl.ds(s * CHUNKS_PER_SUB, IDXG)], dst_v.at[0], sem_i)

    def fill_body(i, carry):
      for t in range(F // 16):
        rows_v[i, pl.ds(t * 16, 16)] = jnp.zeros((16,), jnp.float32)
      return carry
    lax.fori_loop(0, CHUNK, fill_body, 0)
    for t in range(CHUNK // 16):
      ones_v[pl.ds(t * 16, 16)] = jnp.ones((16,), jnp.float32)
    def zfill_body(i, carry):
      cntb_v[pl.ds(i * 16, 16)] = jnp.zeros((16,), jnp.float32)
      return carry
    lax.fori_loop(0, ROWS_PER_SUB // 16, zfill_body, 0)

    hz = [pltpu.async_copy(rows_v, accum.at[pl.ds(base + t * CHUNK, CHUNK)],
                           sem_s)
          for t in range(BLOCKS_PER_SUB)]
    hz.append(pltpu.async_copy(cntb_v, cnt_sp.at[pl.ds(base, ROWS_PER_SUB)],
                               sem_s))
    pltpu.make_async_copy(
        src_hbm.at[c, pl.ds(s * CHUNKS_PER_SUB, IDXG)],
        src_v.at[0], sem_i).wait()
    pltpu.make_async_copy(
        dst_hbm.at[c, pl.ds(s * CHUNKS_PER_SUB, IDXG)],
        dst_v.at[0], sem_i).wait()
    pltpu.async_copy(h_hbm.at[src_v.at[0, 0]], bufs[0], sem)
    for h_ in hz:
      h_.wait()

    plsc.subcore_barrier()

    ngroups = CHUNKS_PER_SUB // IDXG

    def idx_load(g, p):
      pltpu.async_copy(
          src_hbm.at[c, pl.ds(s * CHUNKS_PER_SUB + g * IDXG, IDXG)],
          src_v.at[p], sem_i)
      pltpu.async_copy(
          dst_hbm.at[c, pl.ds(s * CHUNKS_PER_SUB + g * IDXG, IDXG)],
          dst_v.at[p], sem_i)

    def idx_drain(p):
      pltpu.make_async_copy(
          src_hbm.at[c, pl.ds(s * CHUNKS_PER_SUB, IDXG)],
          src_v.at[p], sem_i).wait()
      pltpu.make_async_copy(
          dst_hbm.at[c, pl.ds(s * CHUNKS_PER_SUB, IDXG)],
          dst_v.at[p], sem_i).wait()

    def chunk_body(g, carry):
      p = lax.rem(g, 2)
      @pl.when(g < ngroups - 1)
      def _():
        idx_load(g + 1, 1 - p)
      hc = [pltpu.async_copy(ones_v, cnt_sp.at[dst_v.at[p, t]], sem_c,
                             add=True)
            for t in range(IDXG)]
      hs = [None] * IDXG
      for t in range(IDXG):
        pltpu.make_async_copy(
            h_hbm.at[src_v.at[p, t]], bufs[t % 2], sem).wait()
        if t >= 1:
          hs[t - 1].wait()
        if t + 1 < IDXG:
          pltpu.async_copy(h_hbm.at[src_v.at[p, t + 1]],
                           bufs[(t + 1) % 2], sem)
        hs[t] = pltpu.async_copy(bufs[t % 2], accum.at[dst_v.at[p, t]],
                                 sem_s, add=True)
      hs[IDXG - 1].wait()
      for t in range(IDXG):
        hc[t].wait()
      @pl.when(g < ngroups - 1)
      def _():
        idx_drain(1 - p)
        pltpu.async_copy(h_hbm.at[src_v.at[1 - p, 0]], bufs[0], sem)
      return carry
    lax.fori_loop(0, ngroups, chunk_body, 0)

    plsc.subcore_barrier()

    pltpu.sync_copy(cnt_sp.at[pl.ds(base, ROWS_PER_SUB)], cntb_v)

    def divide(buf, t):
      def div_group(g, carry2):
        cnt16 = cntb_v[pl.ds(t * CHUNK + g * 16, 16)]
        winv = 1.0 / jnp.maximum(cnt16, 1.0)
        for lane in range(16):
          b = jnp.full((16,), winv[lane], jnp.float32)
          r = g * 16 + lane
          for u in range(F // 16):
            buf[r, pl.ds(u * 16, 16)] = buf[r, pl.ds(u * 16, 16)] * b
        return carry2
      lax.fori_loop(0, CHUNK // 16, div_group, 0)

    hin = pltpu.async_copy(accum.at[pl.ds(base, CHUNK)], bufs[0], sem)
    hout = [None] * BLOCKS_PER_SUB
    for t in range(BLOCKS_PER_SUB):
      hin.wait()
      if t >= 1:
        hout[t - 1].wait()
      if t + 1 < BLOCKS_PER_SUB:
        hin = pltpu.async_copy(
            accum.at[pl.ds(base + (t + 1) * CHUNK, CHUNK)],
            bufs[(t + 1) % 2], sem)
      divide(bufs[t % 2], t)
      hout[t] = pltpu.async_copy(
          bufs[t % 2], means_hbm.at[c, pl.ds(base + t * CHUNK, CHUNK)], sem_s)
    hout[BLOCKS_PER_SUB - 1].wait()

  return k(h, src, dst)


def _tc_body(m_ref, w0, b0, w1, b1, w2, b2, w3, b3, w4, b4, o_ref):
  hf = m_ref[0] + m_ref[1]
  x = jnp.tanh(jnp.dot(hf, w0[...], preferred_element_type=jnp.float32)
               + b0[...])
  x = jnp.maximum(jnp.dot(x, w1[...], preferred_element_type=jnp.float32)
                  + b1[...], 0.0)
  x = jnp.maximum(jnp.dot(x, w2[...], preferred_element_type=jnp.float32)
                  + b2[...], 0.0)
  x = jnp.maximum(jnp.dot(x, w3[...], preferred_element_type=jnp.float32)
                  + b3[...], 0.0)
  o_ref[...] = jnp.dot(x, w4[...], preferred_element_type=jnp.float32) + b4[...]


def _tc_mlp(means, W0, b0, W1, b1, W2, b2, W3, b3, W4, b4):
  rows = 1024
  grid = (NFP // rows,)
  wspec = lambda shape: pl.BlockSpec(shape, lambda i: (0, 0))
  return pl.pallas_call(
      _tc_body,
      grid=grid,
      in_specs=[
          pl.BlockSpec((NC, rows, F), lambda i: (0, i, 0)),
          wspec((F, F)), wspec((1, F)),
          wspec((F, F)), wspec((1, F)),
          wspec((F, F)), wspec((1, F)),
          wspec((F, F)), wspec((1, F)),
          wspec((F, 1)), wspec((1, 1)),
      ],
      out_specs=pl.BlockSpec((rows, 1), lambda i: (i, 0)),
      out_shape=jax.ShapeDtypeStruct((NFP, 1), jnp.float32),
  )(means, W0, b0.reshape(1, F), W1, b1.reshape(1, F),
    W2, b2.reshape(1, F), W3, b3.reshape(1, F), W4, b4.reshape(1, 1))


def kernel(h, edge_index_0, edge_index_1,
           W0, b0, W1, b1, W2, b2, W3, b3, W4, b4):
  pad = EP - E
  pad_src = (jnp.arange(pad, dtype=jnp.int32) * 131) % N_ATOM
  pad_dst = N_FRAG + (jnp.arange(pad, dtype=jnp.int32) % (NFP - N_FRAG))
  src = jnp.stack([
      jnp.concatenate([edge_index_0[0], pad_src]),
      jnp.concatenate([edge_index_1[0], pad_src]),
  ]).reshape(2, EP // CHUNK, CHUNK)
  dst = jnp.stack([
      jnp.concatenate([edge_index_0[1], pad_dst]),
      jnp.concatenate([edge_index_1[1], pad_dst]),
  ]).reshape(2, EP // CHUNK, CHUNK)
  means = _sc_agg(h, src, dst)
  out = _tc_mlp(means, W0, b0, W1, b1, W2, b2, W3, b3, W4, b4)
  return out[:N_FRAG]


# speedup vs baseline: 1.0278x; 1.0200x over previous
import functools

import jax
import jax.numpy as jnp
from jax import lax
from jax.experimental import pallas as pl
from jax.experimental.pallas import tpu as pltpu
from jax.experimental.pallas import tpu_sc as plsc

N_ATOM = 10000
N_FRAG = 10000
E = 160000
F = 128

NC = 2
NS = 16
CHUNK = 128
IDXG = 16
CHUNKS_PER_SUB = 80
EP = NS * CHUNK * CHUNKS_PER_SUB
NFP = 10240
ROWS_PER_SUB = NFP // NS
BLOCKS_PER_SUB = ROWS_PER_SUB // CHUNK


def _sc_agg(h, src, dst):
  mesh = plsc.VectorSubcoreMesh(core_axis_name="c", subcore_axis_name="s")

  @functools.partial(
      pl.kernel,
      mesh=mesh,
      out_type=jax.ShapeDtypeStruct((NC, NFP, F), jnp.float32),
      scratch_types=[
          pltpu.VMEM((2, IDXG, CHUNK), jnp.int32),
          pltpu.VMEM((2, IDXG, CHUNK), jnp.int32),
          pltpu.VMEM((CHUNK, F), jnp.float32),
          pltpu.VMEM((CHUNK, F), jnp.float32),
          pltpu.VMEM((CHUNK,), jnp.float32),
          pltpu.VMEM((ROWS_PER_SUB,), jnp.float32),
          pltpu.VMEM_SHARED((NFP, F), jnp.float32),
          pltpu.VMEM_SHARED((NFP,), jnp.float32),
          pltpu.SemaphoreType.DMA,
          pltpu.SemaphoreType.DMA,
          pltpu.SemaphoreType.DMA,
          pltpu.SemaphoreType.DMA,
      ],
  )
  def k(h_hbm, src_hbm, dst_hbm, means_hbm,
        src_v, dst_v, rows_v, rows_w, ones_v, cntb_v, accum, cnt_sp,
        sem, sem_s, sem_c, sem_i):
    c = lax.axis_index("c")
    s = lax.axis_index("s")
    base = s * ROWS_PER_SUB
    bufs = (rows_w, rows_v)

    pltpu.async_copy(
        src_hbm.at[c, pl.ds(s * CHUNKS_PER_SUB, IDXG)], src_v.at[0], sem_i)
    pltpu.async_copy(
        dst_hbm.at[c, pl.ds(s * CHUNKS_PER_SUB, IDXG)], dst_v.at[0], sem_i)

    def fill_body(i, carry):
      for t in range(F // 16):
        rows_v[i, pl.ds(t * 16, 16)] = jnp.zeros((16,), jnp.float32)
      return carry
    lax.fori_loop(0, CHUNK, fill_body, 0)
    for t in range(CHUNK // 16):
      ones_v[pl.ds(t * 16, 16)] = jnp.ones((16,), jnp.float32)
    def zfill_body(i, carry):
      cntb_v[pl.ds(i * 16, 16)] = jnp.zeros((16,), jnp.float32)
      return carry
    lax.fori_loop(0, ROWS_PER_SUB // 16, zfill_body, 0)

    hz = [pltpu.async_copy(rows_v, accum.at[pl.ds(base + t * CHUNK, CHUNK)],
                           sem_s)
          for t in range(BLOCKS_PER_SUB)]
    hz.append(pltpu.async_copy(cntb_v, cnt_sp.at[pl.ds(base, ROWS_PER_SUB)],
                               sem_s))
    pltpu.make_async_copy(
        src_hbm.at[c, pl.ds(s * CHUNKS_PER_SUB, IDXG)],
        src_v.at[0], sem_i).wait()
    pltpu.make_async_copy(
        dst_hbm.at[c, pl.ds(s * CHUNKS_PER_SUB, IDXG)],
        dst_v.at[0], sem_i).wait()
    pltpu.async_copy(h_hbm.at[src_v.at[0, 0]], bufs[0], sem)
    for h_ in hz:
      h_.wait()

    plsc.subcore_barrier()

    ngroups = CHUNKS_PER_SUB // IDXG

    def idx_load(g, p):
      pltpu.async_copy(
          src_hbm.at[c, pl.ds(s * CHUNKS_PER_SUB + g * IDXG, IDXG)],
          src_v.at[p], sem_i)
      pltpu.async_copy(
          dst_hbm.at[c, pl.ds(s * CHUNKS_PER_SUB + g * IDXG, IDXG)],
          dst_v.at[p], sem_i)

    def idx_drain(p):
      pltpu.make_async_copy(
          src_hbm.at[c, pl.ds(s * CHUNKS_PER_SUB, IDXG)],
          src_v.at[p], sem_i).wait()
      pltpu.make_async_copy(
          dst_hbm.at[c, pl.ds(s * CHUNKS_PER_SUB, IDXG)],
          dst_v.at[p], sem_i).wait()

    def chunk_body(g, carry):
      p = lax.rem(g, 2)
      @pl.when(g < ngroups - 1)
      def _():
        idx_load(g + 1, 1 - p)
      hc = [pltpu.async_copy(ones_v, cnt_sp.at[dst_v.at[p, t]], sem_c,
                             add=True)
            for t in range(IDXG)]
      hs = [None] * IDXG
      for t in range(IDXG):
        pltpu.make_async_copy(
            h_hbm.at[src_v.at[p, t]], bufs[t % 2], sem).wait()
        if t >= 1:
          hs[t - 1].wait()
        if t + 1 < IDXG:
          pltpu.async_copy(h_hbm.at[src_v.at[p, t + 1]],
                           bufs[(t + 1) % 2], sem)
        hs[t] = pltpu.async_copy(bufs[t % 2], accum.at[dst_v.at[p, t]],
                                 sem_s, add=True)
      hs[IDXG - 1].wait()
      for t in range(IDXG):
        hc[t].wait()
      @pl.when(g < ngroups - 1)
      def _():
        idx_drain(1 - p)
        pltpu.async_copy(h_hbm.at[src_v.at[1 - p, 0]], bufs[0], sem)
      return carry
    lax.fori_loop(0, ngroups, chunk_body, 0)

    plsc.subcore_barrier()

    pltpu.sync_copy(cnt_sp.at[pl.ds(base, ROWS_PER_SUB)], cntb_v)

    def divide(buf, t):
      def div_group(g, carry2):
        cnt16 = cntb_v[pl.ds(t * CHUNK + g * 16, 16)]
        winv = 1.0 / jnp.maximum(cnt16, 1.0)
        for lane in range(16):
          b = jnp.full((16,), winv[lane], jnp.float32)
          r = g * 16 + lane
          for u in range(F // 16):
            buf[r, pl.ds(u * 16, 16)] = buf[r, pl.ds(u * 16, 16)] * b
        return carry2
      lax.fori_loop(0, CHUNK // 16, div_group, 0)

    hin = pltpu.async_copy(accum.at[pl.ds(base, CHUNK)], bufs[0], sem)
    hout = [None] * BLOCKS_PER_SUB
    for t in range(BLOCKS_PER_SUB):
      hin.wait()
      if t >= 1:
        hout[t - 1].wait()
      if t + 1 < BLOCKS_PER_SUB:
        hin = pltpu.async_copy(
            accum.at[pl.ds(base + (t + 1) * CHUNK, CHUNK)],
            bufs[(t + 1) % 2], sem)
      divide(bufs[t % 2], t)
      hout[t] = pltpu.async_copy(
          bufs[t % 2], means_hbm.at[c, pl.ds(base + t * CHUNK, CHUNK)], sem_s)
    hout[BLOCKS_PER_SUB - 1].wait()

  return k(h, src, dst)


def _tc_body(m_ref, w0, b0, w1, b1, w2, b2, w3, b3, w4, b4, o_ref):
  hf = m_ref[0] + m_ref[1]
  x = jnp.tanh(jnp.dot(hf, w0[...], preferred_element_type=jnp.float32)
               + b0[...])
  x = jnp.maximum(jnp.dot(x, w1[...], preferred_element_type=jnp.float32)
                  + b1[...], 0.0)
  x = jnp.maximum(jnp.dot(x, w2[...], preferred_element_type=jnp.float32)
                  + b2[...], 0.0)
  x = jnp.maximum(jnp.dot(x, w3[...], preferred_element_type=jnp.float32)
                  + b3[...], 0.0)
  o_ref[...] = jnp.dot(x, w4[...], preferred_element_type=jnp.float32) + b4[...]


def _tc_mlp(means, W0, b0, W1, b1, W2, b2, W3, b3, W4, b4):
  rows = 1024
  grid = (NFP // rows,)
  wspec = lambda shape: pl.BlockSpec(shape, lambda i: (0, 0))
  return pl.pallas_call(
      _tc_body,
      grid=grid,
      in_specs=[
          pl.BlockSpec((NC, rows, F), lambda i: (0, i, 0)),
          wspec((F, F)), wspec((1, F)),
          wspec((F, F)), wspec((1, F)),
          wspec((F, F)), wspec((1, F)),
          wspec((F, F)), wspec((1, F)),
          wspec((F, 1)), wspec((1, 1)),
      ],
      out_specs=pl.BlockSpec((rows, 1), lambda i: (i, 0)),
      out_shape=jax.ShapeDtypeStruct((NFP, 1), jnp.float32),
  )(means, W0, b0.reshape(1, F), W1, b1.reshape(1, F),
    W2, b2.reshape(1, F), W3, b3.reshape(1, F), W4, b4.reshape(1, 1))


def kernel(h, edge_index_0, edge_index_1,
           W0, b0, W1, b1, W2, b2, W3, b3, W4, b4):
  pad = EP - E
  pad_src = (jnp.arange(pad, dtype=jnp.int32) * 131) % N_ATOM
  pad_dst = N_FRAG + (jnp.arange(pad, dtype=jnp.int32) % (NFP - N_FRAG))
  src = jnp.stack([
      jnp.concatenate([edge_index_0[0], pad_src]),
      jnp.concatenate([edge_index_1[0], pad_src]),
  ]).reshape(2, EP // CHUNK, CHUNK)
  dst = jnp.stack([
      jnp.concatenate([edge_index_0[1], pad_dst]),
      jnp.concatenate([edge_index_1[1], pad_dst]),
  ]).reshape(2, EP // CHUNK, CHUNK)
  means = _sc_agg(h, src, dst)
  out = _tc_mlp(means, W0, b0, W1, b1, W2, b2, W3, b3, W4, b4)
  return out[:N_FRAG]
